# Initial kernel scaffold; baseline (speedup 1.0000x reference)
import jax
import jax.numpy as jnp
from jax.experimental import pallas as pl


def kernel(x, edge_index, W1, b1, W2, b2):
    raise NotImplementedError("write your pallas kernel here")


# speedup vs baseline: 36.0598x; 36.0598x over previous
import jax
import jax.numpy as jnp
import numpy as np
from jax import lax
from jax.experimental import pallas as pl
from jax.experimental.pallas import tpu as pltpu
from jax.experimental.pallas import tpu_sc as plsc

N = 10000
E = 320000
IN = 128
HID = 256
OUT = 64

NC = 2
NS = 16
NW = NC * NS
K = 128
CPT = 80
E_PAD = NW * CPT * K
N_PAD = 10240
RPS = N_PAD // NS

_mesh = plsc.VectorSubcoreMesh(core_axis_name="c", subcore_axis_name="s")


def _fill_rows(ref, nrows, width, value):
  def body(i, _):
    for k in range(width // 16):
      ref[i, pl.ds(k * 16, 16)] = jnp.full((16,), value, jnp.float32)
    return 0
  lax.fori_loop(0, nrows, body, 0, unroll=2)


def _make_degree_kernel():
  def body(dst_hbm, out_hbm, dst_v, deg_v):
    c = lax.axis_index("c")
    s = lax.axis_index("s")
    wid = c * NS + s
    pltpu.sync_copy(dst_hbm.at[pl.ds(wid * CPT, CPT)], dst_v)

    def zero(i, _):
      deg_v[pl.ds(i * 16, 16)] = jnp.zeros((16,), jnp.float32)
      return 0

    lax.fori_loop(0, N_PAD // 16, zero, 0, unroll=4)
    ones = jnp.ones((16,), jnp.float32)

    def step(r, _):
      for k in range(K // 16):
        idx = dst_v[r, pl.ds(k * 16, 16)]
        plsc.addupdate_scatter(deg_v, [idx], ones)
      return 0

    lax.fori_loop(0, CPT, step, 0)
    pltpu.sync_copy(deg_v, out_hbm.at[c, s])

  return pl.kernel(
      body,
      out_type=jax.ShapeDtypeStruct((NC, NS, N_PAD), jnp.float32),
      mesh=_mesh,
      compiler_params=pltpu.CompilerParams(needs_layout_passes=False),
      scratch_types=[
          pltpu.VMEM((CPT, K), jnp.int32),
          pltpu.VMEM((N_PAD,), jnp.float32),
      ],
  )


def _make_spmm_kernel(D):
  def body(table_hbm, src_hbm, dst_hbm, out_hbm,
           src_v, dst_v, rows_v, acc_sh,
           gs0, gs1, gs2, gs3, ss0, ss1, ss2, ss3):
    c = lax.axis_index("c")
    s = lax.axis_index("s")
    wid = c * NS + s
    gsems = (gs0, gs1, gs2, gs3)
    ssems = (ss0, ss1, ss2, ss3)
    pltpu.sync_copy(src_hbm.at[pl.ds(wid * CPT, CPT)], src_v)
    pltpu.sync_copy(dst_hbm.at[pl.ds(wid * CPT, CPT)], dst_v)
    _fill_rows(rows_v.at[0], K, D, 0.0)
    for t in range(RPS // K):
      pltpu.sync_copy(rows_v.at[0], acc_sh.at[pl.ds(s * RPS + t * K, K)])
    for u in range(4):
      pltpu.async_copy(table_hbm.at[src_v.at[u]], rows_v.at[u], gsems[u])
    plsc.subcore_barrier()

    def step(jj, _):
      for u in range(4):
        j = jj * 4 + u
        pltpu.make_async_copy(table_hbm.at[src_v.at[j]],
                              rows_v.at[u], gsems[u]).wait()
        pltpu.async_copy(rows_v.at[u], acc_sh.at[dst_v.at[j]], ssems[u],
                         add=True)
        u2 = (u + 2) % 4

        @pl.when(jnp.logical_and(j >= 2, j + 2 < CPT))
        def _():
          pltpu.make_async_copy(rows_v.at[u2], acc_sh.at[dst_v.at[0]],
                                ssems[u2]).wait()
          pltpu.async_copy(table_hbm.at[src_v.at[j + 2]], rows_v.at[u2],
                           gsems[u2])
      return 0

    lax.fori_loop(0, CPT // 4, step, 0)
    for u in range(4):
      pltpu.make_async_copy(rows_v.at[u], acc_sh.at[dst_v.at[0]],
                            ssems[u]).wait()
    plsc.subcore_barrier()
    pltpu.sync_copy(acc_sh.at[pl.ds(s * RPS, RPS)],
                    out_hbm.at[c, pl.ds(s * RPS, RPS)])

  return pl.kernel(
      body,
      out_type=jax.ShapeDtypeStruct((NC, N_PAD, D), jnp.float32),
      mesh=_mesh,
      compiler_params=pltpu.CompilerParams(use_tc_tiling_on_sc=False),
      scratch_types=[
          pltpu.VMEM((CPT, K), jnp.int32),
          pltpu.VMEM((CPT, K), jnp.int32),
          pltpu.VMEM((4, K, D), jnp.float32),
          pltpu.VMEM_SHARED((N_PAD, D), jnp.float32),
          pltpu.SemaphoreType.DMA,
          pltpu.SemaphoreType.DMA,
          pltpu.SemaphoreType.DMA,
          pltpu.SemaphoreType.DMA,
          pltpu.SemaphoreType.DMA,
          pltpu.SemaphoreType.DMA,
          pltpu.SemaphoreType.DMA,
          pltpu.SemaphoreType.DMA,
      ],
  )


_degree = _make_degree_kernel()
_spmm = _make_spmm_kernel(OUT)

_BLK = 1024
_GRID = N_PAD // _BLK

_PAD_IDX = N + np.arange(E_PAD - E, dtype=np.int32) % (N_PAD - N)


def _scale_body(degp_ref, x_ref, xsa_ref, xsb_ref, dinv_ref):
  deg = jnp.sum(degp_ref[:], axis=0)[:, None] + 1.0
  dinv = lax.rsqrt(deg)
  dinv_ref[:] = jnp.broadcast_to(dinv, dinv_ref.shape)
  xs = x_ref[:] * dinv
  xsa_ref[:] = xs[:, :OUT]
  xsb_ref[:] = xs[:, OUT:]


_scale = pl.pallas_call(
    _scale_body,
    grid=(_GRID,),
    in_specs=[
        pl.BlockSpec((NW, _BLK), lambda i: (0, i)),
        pl.BlockSpec((_BLK, IN), lambda i: (i, 0)),
    ],
    out_specs=[
        pl.BlockSpec((_BLK, OUT), lambda i: (i, 0)),
        pl.BlockSpec((_BLK, OUT), lambda i: (i, 0)),
        pl.BlockSpec((_BLK, 8), lambda i: (i, 0)),
    ],
    out_shape=[
        jax.ShapeDtypeStruct((N_PAD, OUT), jnp.float32),
        jax.ShapeDtypeStruct((N_PAD, OUT), jnp.float32),
        jax.ShapeDtypeStruct((N_PAD, 8), jnp.float32),
    ],
)


def _dense_body(acca_ref, accb_ref, xsa_ref, xsb_ref, dinv_ref,
                w1_ref, b1_ref, w2_ref, ts_ref):
  dinv = dinv_ref[:, 0:1]
  agga = (acca_ref[0] + acca_ref[1] + xsa_ref[:]) * dinv
  aggb = (accb_ref[0] + accb_ref[1] + xsb_ref[:]) * dinv
  h = (jnp.dot(agga, w1_ref[:OUT], preferred_element_type=jnp.float32)
       + jnp.dot(aggb, w1_ref[OUT:], preferred_element_type=jnp.float32)
       + b1_ref[:])
  h = jnp.maximum(h, 0.0)
  t = jnp.dot(h, w2_ref[:], preferred_element_type=jnp.float32)
  ts_ref[:] = t * dinv


_dense = pl.pallas_call(
    _dense_body,
    grid=(_GRID,),
    in_specs=[
        pl.BlockSpec((NC, _BLK, OUT), lambda i: (0, i, 0)),
        pl.BlockSpec((NC, _BLK, OUT), lambda i: (0, i, 0)),
        pl.BlockSpec((_BLK, OUT), lambda i: (i, 0)),
        pl.BlockSpec((_BLK, OUT), lambda i: (i, 0)),
        pl.BlockSpec((_BLK, 8), lambda i: (i, 0)),
        pl.BlockSpec((IN, HID), lambda i: (0, 0)),
        pl.BlockSpec((1, HID), lambda i: (0, 0)),
        pl.BlockSpec((HID, OUT), lambda i: (0, 0)),
    ],
    out_specs=pl.BlockSpec((_BLK, OUT), lambda i: (i, 0)),
    out_shape=jax.ShapeDtypeStruct((N_PAD, OUT), jnp.float32),
)


def _final_body(acc_ref, ts_ref, dinv_ref, b2_ref, out_ref):
  dinv = dinv_ref[:, 0:1]
  out_ref[:] = (acc_ref[0] + acc_ref[1] + ts_ref[:]) * dinv + b2_ref[:]


_final = pl.pallas_call(
    _final_body,
    grid=(_GRID,),
    in_specs=[
        pl.BlockSpec((NC, _BLK, OUT), lambda i: (0, i, 0)),
        pl.BlockSpec((_BLK, OUT), lambda i: (i, 0)),
        pl.BlockSpec((_BLK, 8), lambda i: (i, 0)),
        pl.BlockSpec((1, OUT), lambda i: (0, 0)),
    ],
    out_specs=pl.BlockSpec((_BLK, OUT), lambda i: (i, 0)),
    out_shape=jax.ShapeDtypeStruct((N_PAD, OUT), jnp.float32),
)


@jax.jit
def kernel(x, edge_index, W1, b1, W2, b2):
  src = edge_index[0]
  dst = edge_index[1]
  pad = jnp.asarray(_PAD_IDX)
  srcp = jnp.concatenate([src, pad]).reshape(NW * CPT, K)
  dstp = jnp.concatenate([dst, pad]).reshape(NW * CPT, K)
  x_pad = jnp.pad(x, ((0, N_PAD - N), (0, 0)))

  degp = _degree(dstp).reshape(NW, N_PAD)
  xsa, xsb, dinv = _scale(degp, x_pad)
  acca = _spmm(xsa, srcp, dstp)
  accb = _spmm(xsb, srcp, dstp)
  ts = _dense(acca, accb, xsa, xsb, dinv, W1, b1.reshape(1, HID), W2)
  acc2 = _spmm(ts, srcp, dstp)
  outp = _final(acc2, ts, dinv, b2.reshape(1, OUT))
  return outp[:N]
